# baseline (speedup 1.0000x reference)
import functools

import jax
import jax.numpy as jnp
from jax import lax
from jax.experimental import pallas as pl
from jax.experimental.pallas import tpu as pltpu
from jax.experimental.pallas import tpu_sc as plsc

N = 10000
E = 320000
D = 128

NC = 2
NS = 16
NW = NC * NS
C = 96
NCH = -(-E // (NW * C))
EW = NCH * C
EPAD = NW * EW
NP = N + 8
NT = 10
RPT = N // NT

_mesh = plsc.VectorSubcoreMesh(core_axis_name="c", subcore_axis_name="s")


@functools.partial(
    pl.kernel,
    out_type=[
        jax.ShapeDtypeStruct((NC, N, D), jnp.float32),
        jax.ShapeDtypeStruct((NC, NP), jnp.float32),
    ],
    mesh=_mesh,
    scratch_types=[
        pltpu.VMEM((EW,), jnp.int32),
        pltpu.VMEM((2, C), jnp.int32),
        pltpu.VMEM((2, C, D), jnp.float32),
        pltpu.VMEM((C,), jnp.float32),
        pltpu.VMEM_SHARED((NP, D), jnp.float32),
        pltpu.VMEM_SHARED((NP,), jnp.float32),
        pltpu.SemaphoreType.DMA,
        pltpu.SemaphoreType.DMA,
        pltpu.SemaphoreType.DMA,
        pltpu.SemaphoreType.DMA,
    ],
)
def _sc_scatter(h_hbm, src_hbm, dst_hbm, zrow_hbm, zcnt_hbm,
                out_hbm, cnt_hbm,
                src_v, dst_v, rows_v, ones_v, acc_s, cnt_s,
                sg0, sg1, sd0, sd1):
    cid = lax.axis_index("c")
    sid = lax.axis_index("s")
    wid = sid * NC + cid

    @pl.when(sid < NT)
    def _():
        pltpu.sync_copy(zrow_hbm, acc_s.at[pl.ds(sid * RPT, RPT)])

    @pl.when(sid == 0)
    def _():
        pltpu.sync_copy(zcnt_hbm, cnt_s)

    pltpu.sync_copy(src_hbm.at[pl.ds(wid * EW, EW)], src_v)

    for j in range(C // 16):
        ones_v[pl.ds(j * 16, 16)] = jnp.ones((16,), jnp.float32)

    plsc.subcore_barrier()

    sgs = (sg0, sg1)
    sds = (sd0, sd1)

    def src_chunk(i):
        return src_v.at[pl.ds(pl.multiple_of(i * C, 8), C)]

    def fire(i, b):
        pltpu.async_copy(
            dst_hbm.at[pl.ds(pl.multiple_of(wid * EW + i * C, 8), C)],
            dst_v.at[b], sds[b])
        pltpu.async_copy(h_hbm.at[src_chunk(i)], rows_v.at[b], sgs[b])

    def drain_and_scatter(i, b):
        pltpu.make_async_copy(
            dst_hbm.at[pl.ds(pl.multiple_of(wid * EW + i * C, 8), C)],
            dst_v.at[b], sds[b]).wait()
        pltpu.sync_copy(ones_v, cnt_s.at[dst_v.at[b]], add=True)
        pltpu.make_async_copy(h_hbm.at[src_chunk(i)], rows_v.at[b], sgs[b]).wait()
        pltpu.sync_copy(rows_v.at[b], acc_s.at[dst_v.at[b]], add=True)

    fire(0, 0)

    def body(k, carry):
        i0 = 2 * k
        fire(i0 + 1, 1)
        drain_and_scatter(i0, 0)
        fire(i0 + 2, 0)
        drain_and_scatter(i0 + 1, 1)
        return carry

    lax.fori_loop(0, (NCH - 1) // 2, body, 0)
    if NCH % 2 == 1:
        drain_and_scatter(NCH - 1, 0)
    else:
        fire(NCH - 1, 1)
        drain_and_scatter(NCH - 2, 0)
        drain_and_scatter(NCH - 1, 1)

    plsc.subcore_barrier()

    @pl.when(sid < NT)
    def _():
        pltpu.sync_copy(acc_s.at[pl.ds(sid * RPT, RPT)],
                        out_hbm.at[cid, pl.ds(sid * RPT, RPT)])

    @pl.when(sid == 0)
    def _():
        pltpu.sync_copy(cnt_s, cnt_hbm.at[cid])


def _mm_body(x_ref, w_ref, b_ref, h_ref, z_ref):
    acc = jnp.dot(x_ref[...], w_ref[...],
                  preferred_element_type=jnp.float32) + b_ref[...]
    h_ref[...] = acc[:, :D]
    z_ref[...] = acc[:, D:]


_R = 1000


def _matmul2(x, w_cat, b_cat):
    grid = (N // _R,)
    return pl.pallas_call(
        _mm_body,
        grid=grid,
        in_specs=[
            pl.BlockSpec((_R, D), lambda i: (i, 0)),
            pl.BlockSpec((D, 2 * D), lambda i: (0, 0)),
            pl.BlockSpec((1, 2 * D), lambda i: (0, 0)),
        ],
        out_specs=[
            pl.BlockSpec((_R, D), lambda i: (i, 0)),
            pl.BlockSpec((_R, D), lambda i: (i, 0)),
        ],
        out_shape=[
            jax.ShapeDtypeStruct((N, D), jnp.float32),
            jax.ShapeDtypeStruct((N, D), jnp.float32),
        ],
    )(x, w_cat, b_cat)


def _combine_mm_body(p_ref, cnt_ref, z_ref, w_ref, b_ref, h2_ref, z2_ref):
    cntv = cnt_ref[...]
    tot = cntv[:, 0:1] + cntv[:, 1:2]
    rcp = 1.0 / jnp.maximum(tot, 1.0)
    h1 = jax.nn.relu((p_ref[0] + p_ref[1]) * rcp + z_ref[...])
    acc = jnp.dot(h1, w_ref[...], preferred_element_type=jnp.float32) + b_ref[...]
    h2_ref[...] = acc[:, :D]
    z2_ref[...] = acc[:, D:]


def _combine_matmul(p, cnt, z, w_cat, b_cat):
    grid = (N // _R,)
    return pl.pallas_call(
        _combine_mm_body,
        grid=grid,
        in_specs=[
            pl.BlockSpec((2, _R, D), lambda i: (0, i, 0)),
            pl.BlockSpec((_R, NC), lambda i: (i, 0)),
            pl.BlockSpec((_R, D), lambda i: (i, 0)),
            pl.BlockSpec((D, 2 * D), lambda i: (0, 0)),
            pl.BlockSpec((1, 2 * D), lambda i: (0, 0)),
        ],
        out_specs=[
            pl.BlockSpec((_R, D), lambda i: (i, 0)),
            pl.BlockSpec((_R, D), lambda i: (i, 0)),
        ],
        out_shape=[
            jax.ShapeDtypeStruct((N, D), jnp.float32),
            jax.ShapeDtypeStruct((N, D), jnp.float32),
        ],
    )(p, cnt, z, w_cat, b_cat)


def _final_body(p_ref, cnt_ref, z_ref, out_ref):
    cntv = cnt_ref[...]
    tot = cntv[:, 0:1] + cntv[:, 1:2]
    rcp = 1.0 / jnp.maximum(tot, 1.0)
    out_ref[...] = (p_ref[0] + p_ref[1]) * rcp + z_ref[...]


def _final_combine(p, cnt, z):
    grid = (N // _R,)
    return pl.pallas_call(
        _final_body,
        grid=grid,
        in_specs=[
            pl.BlockSpec((2, _R, D), lambda i: (0, i, 0)),
            pl.BlockSpec((_R, NC), lambda i: (i, 0)),
            pl.BlockSpec((_R, D), lambda i: (i, 0)),
        ],
        out_specs=pl.BlockSpec((_R, D), lambda i: (i, 0)),
        out_shape=jax.ShapeDtypeStruct((N, D), jnp.float32),
    )(p, cnt, z)


def kernel(x, edge_index, W1, lin1_W, lin1_b, W2, lin2_W, lin2_b):
    src = jnp.concatenate(
        [edge_index[0].astype(jnp.int32), jnp.zeros((EPAD - E,), jnp.int32)])
    dst = jnp.concatenate(
        [edge_index[1].astype(jnp.int32), jnp.full((EPAD - E,), N, jnp.int32)])
    zrow = jnp.zeros((RPT, D), jnp.float32)
    zcnt = jnp.zeros((NP,), jnp.float32)

    w1c = jnp.concatenate([W1, lin1_W], axis=1)
    b1c = jnp.concatenate([jnp.zeros((D,), jnp.float32), lin1_b]).reshape(1, 2 * D)
    w2c = jnp.concatenate([W2, lin2_W], axis=1)
    b2c = jnp.concatenate([jnp.zeros((D,), jnp.float32), lin2_b]).reshape(1, 2 * D)

    h1, z1 = _matmul2(x, w1c, b1c)
    p1, cnt = _sc_scatter(h1, src, dst, zrow, zcnt)
    cnt_t = jnp.transpose(cnt[:, :N])
    h2, z2 = _combine_matmul(p1, cnt_t, z1, w2c, b2c)
    p2, _cnt2 = _sc_scatter(h2, src, dst, zrow, zcnt)
    return _final_combine(p2, cnt_t, z2)


# speedup vs baseline: 1.0886x; 1.0886x over previous
import functools

import jax
import jax.numpy as jnp
from jax import lax
from jax.experimental import pallas as pl
from jax.experimental.pallas import tpu as pltpu
from jax.experimental.pallas import tpu_sc as plsc

N = 10000
E = 320000
D = 128

NC = 2
NS = 16
NW = NC * NS
C = 96
N0 = 133
N1 = 77
EPAD = NS * (N0 + N1) * C
SC0E = NS * N0 * C
NP = N + 8
NT = 10
RPT = N // NT

_mesh = plsc.VectorSubcoreMesh(core_axis_name="c", subcore_axis_name="s")


@functools.partial(
    pl.kernel,
    out_type=[
        jax.ShapeDtypeStruct((NC, N, D), jnp.float32),
        jax.ShapeDtypeStruct((NC, NP), jnp.float32),
    ],
    mesh=_mesh,
    scratch_types=[
        pltpu.VMEM((N0 * C,), jnp.int32),
        pltpu.VMEM((2, C), jnp.int32),
        pltpu.VMEM((2, C, D), jnp.float32),
        pltpu.VMEM((C,), jnp.float32),
        pltpu.VMEM_SHARED((NP, D), jnp.float32),
        pltpu.VMEM_SHARED((NP,), jnp.float32),
        pltpu.SemaphoreType.DMA,
        pltpu.SemaphoreType.DMA,
        pltpu.SemaphoreType.DMA,
        pltpu.SemaphoreType.DMA,
    ],
)
def _sc_scatter(h_hbm, src_hbm, dst_hbm, zrow_hbm, zcnt_hbm,
                out_hbm, cnt_hbm,
                src_v, dst_v, rows_v, ones_v, acc_s, cnt_s,
                sg0, sg1, sd0, sd1):
    cid = lax.axis_index("c")
    sid = lax.axis_index("s")
    base_w = lax.select(cid == 0, sid * (N0 * C), SC0E + sid * (N1 * C))
    ncw = lax.select(cid == 0, N0, N1)

    @pl.when(sid < NT)
    def _():
        pltpu.sync_copy(zrow_hbm, acc_s.at[pl.ds(sid * RPT, RPT)])

    @pl.when(sid == 0)
    def _():
        pltpu.sync_copy(zcnt_hbm, cnt_s)

    pltpu.sync_copy(src_hbm.at[pl.ds(pl.multiple_of(base_w, 8), N1 * C)],
                    src_v.at[pl.ds(0, N1 * C)])

    @pl.when(cid == 0)
    def _():
        pltpu.sync_copy(
            src_hbm.at[pl.ds(pl.multiple_of(base_w + N1 * C, 8), (N0 - N1) * C)],
            src_v.at[pl.ds(N1 * C, (N0 - N1) * C)])

    for j in range(C // 16):
        ones_v[pl.ds(j * 16, 16)] = jnp.ones((16,), jnp.float32)

    plsc.subcore_barrier()

    sgs = (sg0, sg1)
    sds = (sd0, sd1)

    def src_chunk(i):
        return src_v.at[pl.ds(pl.multiple_of(i * C, 8), C)]

    def fire(i, b):
        pltpu.async_copy(
            dst_hbm.at[pl.ds(pl.multiple_of(base_w + i * C, 8), C)],
            dst_v.at[b], sds[b])
        pltpu.async_copy(h_hbm.at[src_chunk(i)], rows_v.at[b], sgs[b])

    def drain_and_scatter(i, b):
        pltpu.make_async_copy(
            dst_hbm.at[pl.ds(pl.multiple_of(base_w + i * C, 8), C)],
            dst_v.at[b], sds[b]).wait()
        pltpu.sync_copy(ones_v, cnt_s.at[dst_v.at[b]], add=True)
        pltpu.make_async_copy(h_hbm.at[src_chunk(i)], rows_v.at[b], sgs[b]).wait()
        pltpu.sync_copy(rows_v.at[b], acc_s.at[dst_v.at[b]], add=True)

    fire(0, 0)

    def body(k, carry):
        i0 = 2 * k
        fire(i0 + 1, 1)
        drain_and_scatter(i0, 0)
        fire(i0 + 2, 0)
        drain_and_scatter(i0 + 1, 1)
        return carry

    lax.fori_loop(0, lax.select(cid == 0, (N0 - 1) // 2, (N1 - 1) // 2), body, 0)
    drain_and_scatter(ncw - 1, 0)

    plsc.subcore_barrier()

    @pl.when(sid < NT)
    def _():
        pltpu.sync_copy(acc_s.at[pl.ds(sid * RPT, RPT)],
                        out_hbm.at[cid, pl.ds(sid * RPT, RPT)])

    @pl.when(sid == 0)
    def _():
        pltpu.sync_copy(cnt_s, cnt_hbm.at[cid])


def _mm_body(x_ref, w_ref, b_ref, h_ref, z_ref):
    acc = jnp.dot(x_ref[...], w_ref[...],
                  preferred_element_type=jnp.float32) + b_ref[...]
    h_ref[...] = acc[:, :D]
    z_ref[...] = acc[:, D:]


_R = 1000


def _matmul2(x, w_cat, b_cat):
    grid = (N // _R,)
    return pl.pallas_call(
        _mm_body,
        grid=grid,
        in_specs=[
            pl.BlockSpec((_R, D), lambda i: (i, 0)),
            pl.BlockSpec((D, 2 * D), lambda i: (0, 0)),
            pl.BlockSpec((1, 2 * D), lambda i: (0, 0)),
        ],
        out_specs=[
            pl.BlockSpec((_R, D), lambda i: (i, 0)),
            pl.BlockSpec((_R, D), lambda i: (i, 0)),
        ],
        out_shape=[
            jax.ShapeDtypeStruct((N, D), jnp.float32),
            jax.ShapeDtypeStruct((N, D), jnp.float32),
        ],
    )(x, w_cat, b_cat)


def _combine_mm_body(p_ref, cnt_ref, z_ref, w_ref, b_ref, h2_ref, z2_ref):
    cntv = cnt_ref[...]
    tot = cntv[:, 0:1] + cntv[:, 1:2]
    rcp = 1.0 / jnp.maximum(tot, 1.0)
    h1 = jax.nn.relu((p_ref[0] + p_ref[1]) * rcp + z_ref[...])
    acc = jnp.dot(h1, w_ref[...], preferred_element_type=jnp.float32) + b_ref[...]
    h2_ref[...] = acc[:, :D]
    z2_ref[...] = acc[:, D:]


def _combine_matmul(p, cnt, z, w_cat, b_cat):
    grid = (N // _R,)
    return pl.pallas_call(
        _combine_mm_body,
        grid=grid,
        in_specs=[
            pl.BlockSpec((2, _R, D), lambda i: (0, i, 0)),
            pl.BlockSpec((_R, NC), lambda i: (i, 0)),
            pl.BlockSpec((_R, D), lambda i: (i, 0)),
            pl.BlockSpec((D, 2 * D), lambda i: (0, 0)),
            pl.BlockSpec((1, 2 * D), lambda i: (0, 0)),
        ],
        out_specs=[
            pl.BlockSpec((_R, D), lambda i: (i, 0)),
            pl.BlockSpec((_R, D), lambda i: (i, 0)),
        ],
        out_shape=[
            jax.ShapeDtypeStruct((N, D), jnp.float32),
            jax.ShapeDtypeStruct((N, D), jnp.float32),
        ],
    )(p, cnt, z, w_cat, b_cat)


def _final_body(p_ref, cnt_ref, z_ref, out_ref):
    cntv = cnt_ref[...]
    tot = cntv[:, 0:1] + cntv[:, 1:2]
    rcp = 1.0 / jnp.maximum(tot, 1.0)
    out_ref[...] = (p_ref[0] + p_ref[1]) * rcp + z_ref[...]


def _final_combine(p, cnt, z):
    grid = (N // _R,)
    return pl.pallas_call(
        _final_body,
        grid=grid,
        in_specs=[
            pl.BlockSpec((2, _R, D), lambda i: (0, i, 0)),
            pl.BlockSpec((_R, NC), lambda i: (i, 0)),
            pl.BlockSpec((_R, D), lambda i: (i, 0)),
        ],
        out_specs=pl.BlockSpec((_R, D), lambda i: (i, 0)),
        out_shape=jax.ShapeDtypeStruct((N, D), jnp.float32),
    )(p, cnt, z)


def kernel(x, edge_index, W1, lin1_W, lin1_b, W2, lin2_W, lin2_b):
    src = jnp.concatenate(
        [edge_index[0].astype(jnp.int32), jnp.zeros((EPAD - E,), jnp.int32)])
    dst = jnp.concatenate(
        [edge_index[1].astype(jnp.int32), jnp.full((EPAD - E,), N, jnp.int32)])
    zrow = jnp.zeros((RPT, D), jnp.float32)
    zcnt = jnp.zeros((NP,), jnp.float32)

    w1c = jnp.concatenate([W1, lin1_W], axis=1)
    b1c = jnp.concatenate([jnp.zeros((D,), jnp.float32), lin1_b]).reshape(1, 2 * D)
    w2c = jnp.concatenate([W2, lin2_W], axis=1)
    b2c = jnp.concatenate([jnp.zeros((D,), jnp.float32), lin2_b]).reshape(1, 2 * D)

    h1, z1 = _matmul2(x, w1c, b1c)
    p1, cnt = _sc_scatter(h1, src, dst, zrow, zcnt)
    cnt_t = jnp.transpose(cnt[:, :N])
    h2, z2 = _combine_matmul(p1, cnt_t, z1, w2c, b2c)
    p2, _cnt2 = _sc_scatter(h2, src, dst, zrow, zcnt)
    return _final_combine(p2, cnt_t, z2)
